# Initial kernel scaffold; baseline (speedup 1.0000x reference)
import jax
import jax.numpy as jnp
from jax.experimental import pallas as pl


def kernel(node_features, edge_index, W1, b1, W2, b2):
    raise NotImplementedError("write your pallas kernel here")


# speedup vs baseline: 27.6329x; 27.6329x over previous
import functools

import jax
import jax.numpy as jnp
import numpy as np
from jax import lax
from jax.experimental import pallas as pl
from jax.experimental.pallas import tpu as pltpu
from jax.experimental.pallas import tpu_sc as plsc

NUM_CLASSES = 8

_NC = 2
_NS = 16
_NW = _NC * _NS
_ROW = 128
_KROWS = 8


def _sc_segment_sum(x_ext, src2d, dst2d, zeros_init, n_pad, rows_per_w):
    rows_per_sub = n_pad // _NS
    n_chunks = rows_per_w // _KROWS
    mesh = plsc.VectorSubcoreMesh(core_axis_name="c", subcore_axis_name="s")

    @functools.partial(
        pl.kernel,
        mesh=mesh,
        out_type=jax.ShapeDtypeStruct((2 * n_pad, 8), jnp.float32),
        scratch_types=(
            [pltpu.VMEM((_ROW,), jnp.int32)] * (2 * _KROWS)
            + [pltpu.VMEM((_ROW, 8), jnp.float32)] * 2
            + [pltpu.VMEM_SHARED((n_pad, 8), jnp.float32)]
            + [pltpu.SemaphoreType.DMA] * 3
        ),
        compiler_params=pltpu.CompilerParams(use_tc_tiling_on_sc=False),
    )
    def seg_kernel(x_hbm, src_hbm, dst_hbm, zer_hbm, out_hbm, *rest):
        srcr = rest[0:_KROWS]
        dstr = rest[_KROWS:2 * _KROWS]
        bufs = rest[2 * _KROWS:2 * _KROWS + 2]
        acc_sp = rest[2 * _KROWS + 2]
        sems = rest[2 * _KROWS + 3:2 * _KROWS + 6]
        cid = lax.axis_index("c")
        sid = lax.axis_index("s")
        wid = sid * _NC + cid

        pltpu.sync_copy(zer_hbm.at[pl.ds(sid * rows_per_sub, rows_per_sub)],
                        acc_sp.at[pl.ds(sid * rows_per_sub, rows_per_sub)])
        plsc.subcore_barrier()

        def chunk(g, carry):
            base_row = wid * rows_per_w + g * _KROWS
            loads = []
            for j in range(_KROWS):
                loads.append(pltpu.async_copy(src_hbm.at[base_row + j],
                                              srcr[j], sems[2]))
                loads.append(pltpu.async_copy(dst_hbm.at[base_row + j],
                                              dstr[j], sems[2]))
            for cp in loads:
                cp.wait()
            cp = pltpu.async_copy(x_hbm.at[srcr[0]], bufs[0], sems[0])
            for j in range(_KROWS):
                cp.wait()
                if j + 1 < _KROWS:
                    cp = pltpu.async_copy(x_hbm.at[srcr[j + 1]],
                                          bufs[(j + 1) % 2], sems[(j + 1) % 2])
                pltpu.sync_copy(bufs[j % 2], acc_sp.at[dstr[j]], add=True)
            return carry

        lax.fori_loop(0, n_chunks, chunk, 0)
        plsc.subcore_barrier()

        pltpu.sync_copy(
            acc_sp.at[pl.ds(sid * rows_per_sub, rows_per_sub)],
            out_hbm.at[pl.ds(cid * n_pad + sid * rows_per_sub, rows_per_sub)])

    return seg_kernel(x_ext, src2d, dst2d, zeros_init)


def _tc_head(partials, nfp, w1a, w1b, b1r, w2p, b2p, n_pad, blk):
    grid = (n_pad // blk,)

    def head_kernel(p_ref, nf_ref, w1a_ref, w1b_ref, b1_ref, w2_ref, b2_ref,
                    out_ref):
        s = p_ref[0] + p_ref[1]
        deg = jnp.maximum(s[:, 2:3], 1.0)
        agg4 = s / deg
        nf = nf_ref[...]
        hid = jnp.tanh(
            jnp.dot(nf, w1a_ref[...], preferred_element_type=jnp.float32)
            + jnp.dot(agg4, w1b_ref[...], preferred_element_type=jnp.float32)
            + b1_ref[...])
        o = jnp.dot(hid, w2_ref[...], preferred_element_type=jnp.float32) \
            + b2_ref[...]
        mu = jnp.arctan2(o[:, 0:1], o[:, 1:2])
        ang = jnp.mod(mu, 2.0 * np.pi)
        bin_size = 2.0 * np.pi / NUM_CLASSES
        cid = jnp.clip(jnp.floor(ang / bin_size), 0, NUM_CLASSES - 1)
        cid = cid.astype(jnp.int32)
        col = lax.broadcasted_iota(jnp.int32, (blk, NUM_CLASSES), 1)
        out_ref[...] = jnp.where(col == cid, 0.0, -1000.0)

    return pl.pallas_call(
        head_kernel,
        grid=grid,
        in_specs=[
            pl.BlockSpec((2, blk, 8), lambda i: (0, i, 0)),
            pl.BlockSpec((blk, 4), lambda i: (i, 0)),
            pl.BlockSpec((4, 32), lambda i: (0, 0)),
            pl.BlockSpec((8, 32), lambda i: (0, 0)),
            pl.BlockSpec((1, 32), lambda i: (0, 0)),
            pl.BlockSpec((32, NUM_CLASSES), lambda i: (0, 0)),
            pl.BlockSpec((1, NUM_CLASSES), lambda i: (0, 0)),
        ],
        out_specs=pl.BlockSpec((blk, NUM_CLASSES), lambda i: (i, 0)),
        out_shape=jax.ShapeDtypeStruct((n_pad, NUM_CLASSES), jnp.float32),
    )(partials, nfp, w1a, w1b, b1r, w2p, b2p)


def kernel(node_features, edge_index, W1, b1, W2, b2):
    n = node_features.shape[0]
    e = edge_index.shape[1]
    f32 = jnp.float32

    blk = 2048
    n_pad = ((n + blk - 1) // blk) * blk
    chunk_edges = _NW * _KROWS * _ROW
    e_pad = ((e + chunk_edges - 1) // chunk_edges) * chunk_edges
    r = e_pad // _ROW
    rows_per_w = r // _NW

    ones = jnp.ones((n, 1), f32)
    x_ext = jnp.concatenate([node_features[:, 1:3], ones,
                             jnp.zeros((n, 5), f32)], axis=1)
    x_ext = jnp.pad(x_ext, ((0, n_pad - n), (0, 0)))
    ep = jnp.pad(edge_index, ((0, 0), (0, e_pad - e)), constant_values=n)
    src2d = ep[0].reshape(r, _ROW)
    dst2d = ep[1].reshape(r, _ROW)
    zeros_init = jnp.zeros((n_pad, 8), f32)

    partials = _sc_segment_sum(x_ext, src2d, dst2d, zeros_init,
                               n_pad, rows_per_w)
    partials = partials.reshape(2, n_pad, 8)

    nfp = jnp.pad(node_features, ((0, n_pad - n), (0, 1)))
    w1a = jnp.concatenate([W1[0:3], jnp.zeros((1, 32), f32)], axis=0)
    w1b = jnp.concatenate([W1[3:5], jnp.zeros((6, 32), f32)], axis=0)
    b1r = b1.reshape(1, 32)
    w2p = jnp.concatenate([W2[:, 0:2],
                           jnp.zeros((32, NUM_CLASSES - 2), f32)], axis=1)
    b2p = jnp.concatenate([b2[0:2],
                           jnp.zeros((NUM_CLASSES - 2,), f32)]).reshape(1, -1)

    logits = _tc_head(partials, nfp, w1a, w1b, b1r, w2p, b2p, n_pad, blk)
    return logits[:n]


# speedup vs baseline: 27.6346x; 1.0001x over previous
import functools

import jax
import jax.numpy as jnp
import numpy as np
from jax import lax
from jax.experimental import pallas as pl
from jax.experimental.pallas import tpu as pltpu
from jax.experimental.pallas import tpu_sc as plsc

NUM_CLASSES = 8

_NC = 2
_NS = 16
_NW = _NC * _NS
_ROW = 128
_KROWS = 8


def _sc_segment_sum(x_ext, src2d, dst2d, zeros_init, n_pad, rows_per_w):
    rows_per_sub = n_pad // _NS
    n_chunks = rows_per_w // _KROWS
    mesh = plsc.VectorSubcoreMesh(core_axis_name="c", subcore_axis_name="s")

    @functools.partial(
        pl.kernel,
        mesh=mesh,
        out_type=jax.ShapeDtypeStruct((2 * n_pad, 8), jnp.float32),
        scratch_types=(
            [pltpu.VMEM((_KROWS, _ROW), jnp.int32)] * 2
            + [pltpu.VMEM((_ROW, 8), jnp.float32)] * 2
            + [pltpu.VMEM_SHARED((n_pad, 8), jnp.float32)]
            + [pltpu.SemaphoreType.DMA] * 3
        ),
        compiler_params=pltpu.CompilerParams(use_tc_tiling_on_sc=False),
    )
    def seg_kernel(x_hbm, src_hbm, dst_hbm, zer_hbm, out_hbm,
                   src_v, dst_v, *rest):
        bufs = rest[0:2]
        acc_sp = rest[2]
        sems = rest[3:6]
        cid = lax.axis_index("c")
        sid = lax.axis_index("s")
        wid = sid * _NC + cid

        pltpu.sync_copy(zer_hbm.at[pl.ds(sid * rows_per_sub, rows_per_sub)],
                        acc_sp.at[pl.ds(sid * rows_per_sub, rows_per_sub)])
        plsc.subcore_barrier()

        def chunk(g, carry):
            base_row = wid * rows_per_w + g * _KROWS
            c1 = pltpu.async_copy(src_hbm.at[pl.ds(base_row, _KROWS)],
                                  src_v, sems[2])
            c2 = pltpu.async_copy(dst_hbm.at[pl.ds(base_row, _KROWS)],
                                  dst_v, sems[2])
            c1.wait()
            c2.wait()
            cp = pltpu.async_copy(x_hbm.at[src_v.at[0]], bufs[0], sems[0])
            for j in range(_KROWS):
                cp.wait()
                if j + 1 < _KROWS:
                    cp = pltpu.async_copy(x_hbm.at[src_v.at[j + 1]],
                                          bufs[(j + 1) % 2], sems[(j + 1) % 2])
                pltpu.sync_copy(bufs[j % 2], acc_sp.at[dst_v.at[j]], add=True)
            return carry

        lax.fori_loop(0, n_chunks, chunk, 0)
        plsc.subcore_barrier()

        pltpu.sync_copy(
            acc_sp.at[pl.ds(sid * rows_per_sub, rows_per_sub)],
            out_hbm.at[pl.ds(cid * n_pad + sid * rows_per_sub, rows_per_sub)])

    return seg_kernel(x_ext, src2d, dst2d, zeros_init)


def _tc_head(partials, nfp, w1a, w1b, b1r, w2p, b2p, n_pad, blk):
    grid = (n_pad // blk,)

    def head_kernel(p_ref, nf_ref, w1a_ref, w1b_ref, b1_ref, w2_ref, b2_ref,
                    out_ref):
        s = p_ref[0] + p_ref[1]
        deg = jnp.maximum(s[:, 2:3], 1.0)
        agg4 = s / deg
        nf = nf_ref[...]
        hid = jnp.tanh(
            jnp.dot(nf, w1a_ref[...], preferred_element_type=jnp.float32)
            + jnp.dot(agg4, w1b_ref[...], preferred_element_type=jnp.float32)
            + b1_ref[...])
        o = jnp.dot(hid, w2_ref[...], preferred_element_type=jnp.float32) \
            + b2_ref[...]
        mu = jnp.arctan2(o[:, 0:1], o[:, 1:2])
        ang = jnp.mod(mu, 2.0 * np.pi)
        bin_size = 2.0 * np.pi / NUM_CLASSES
        cid = jnp.clip(jnp.floor(ang / bin_size), 0, NUM_CLASSES - 1)
        cid = cid.astype(jnp.int32)
        col = lax.broadcasted_iota(jnp.int32, (blk, NUM_CLASSES), 1)
        out_ref[...] = jnp.where(col == cid, 0.0, -1000.0)

    return pl.pallas_call(
        head_kernel,
        grid=grid,
        in_specs=[
            pl.BlockSpec((2, blk, 8), lambda i: (0, i, 0)),
            pl.BlockSpec((blk, 4), lambda i: (i, 0)),
            pl.BlockSpec((4, 32), lambda i: (0, 0)),
            pl.BlockSpec((8, 32), lambda i: (0, 0)),
            pl.BlockSpec((1, 32), lambda i: (0, 0)),
            pl.BlockSpec((32, NUM_CLASSES), lambda i: (0, 0)),
            pl.BlockSpec((1, NUM_CLASSES), lambda i: (0, 0)),
        ],
        out_specs=pl.BlockSpec((blk, NUM_CLASSES), lambda i: (i, 0)),
        out_shape=jax.ShapeDtypeStruct((n_pad, NUM_CLASSES), jnp.float32),
    )(partials, nfp, w1a, w1b, b1r, w2p, b2p)


def kernel(node_features, edge_index, W1, b1, W2, b2):
    n = node_features.shape[0]
    e = edge_index.shape[1]
    f32 = jnp.float32

    blk = 2048
    n_pad = ((n + blk - 1) // blk) * blk
    chunk_edges = _NW * _KROWS * _ROW
    e_pad = ((e + chunk_edges - 1) // chunk_edges) * chunk_edges
    r = e_pad // _ROW
    rows_per_w = r // _NW

    ones = jnp.ones((n, 1), f32)
    x_ext = jnp.concatenate([node_features[:, 1:3], ones,
                             jnp.zeros((n, 5), f32)], axis=1)
    x_ext = jnp.pad(x_ext, ((0, n_pad - n), (0, 0)))
    ep = jnp.pad(edge_index, ((0, 0), (0, e_pad - e)), constant_values=n)
    src2d = ep[0].reshape(r, _ROW)
    dst2d = ep[1].reshape(r, _ROW)
    zeros_init = jnp.zeros((n_pad, 8), f32)

    partials = _sc_segment_sum(x_ext, src2d, dst2d, zeros_init,
                               n_pad, rows_per_w)
    partials = partials.reshape(2, n_pad, 8)

    nfp = jnp.pad(node_features, ((0, n_pad - n), (0, 1)))
    w1a = jnp.concatenate([W1[0:3], jnp.zeros((1, 32), f32)], axis=0)
    w1b = jnp.concatenate([W1[3:5], jnp.zeros((6, 32), f32)], axis=0)
    b1r = b1.reshape(1, 32)
    w2p = jnp.concatenate([W2[:, 0:2],
                           jnp.zeros((32, NUM_CLASSES - 2), f32)], axis=1)
    b2p = jnp.concatenate([b2[0:2],
                           jnp.zeros((NUM_CLASSES - 2,), f32)]).reshape(1, -1)

    logits = _tc_head(partials, nfp, w1a, w1b, b1r, w2p, b2p, n_pad, blk)
    return logits[:n]


# speedup vs baseline: 49.9020x; 1.8058x over previous
import functools

import jax
import jax.numpy as jnp
import numpy as np
from jax import lax
from jax.experimental import pallas as pl
from jax.experimental.pallas import tpu as pltpu
from jax.experimental.pallas import tpu_sc as plsc

NUM_CLASSES = 8

_NC = 2
_NS = 16
_NW = _NC * _NS
_ROW = 128
_KROWS = 8


def _sc_segment_sum(x_ext, src2d, dst2d, zeros_init, n_pad, rows_per_w):
    rows_per_sub = n_pad // _NS
    n_chunks = rows_per_w // _KROWS
    mesh = plsc.VectorSubcoreMesh(core_axis_name="c", subcore_axis_name="s")

    @functools.partial(
        pl.kernel,
        mesh=mesh,
        out_type=jax.ShapeDtypeStruct((2 * n_pad, 8), jnp.float32),
        scratch_types=(
            [pltpu.VMEM((_KROWS, _ROW), jnp.int32)] * 2
            + [pltpu.VMEM((_ROW, 8), jnp.float32)] * _KROWS
            + [pltpu.VMEM_SHARED((n_pad, 8), jnp.float32)]
            + [pltpu.SemaphoreType.DMA] * (1 + 2 * _KROWS)
        ),
        compiler_params=pltpu.CompilerParams(use_tc_tiling_on_sc=False),
    )
    def seg_kernel(x_hbm, src_hbm, dst_hbm, zer_hbm, out_hbm,
                   src_v, dst_v, *rest):
        bufs = rest[0:_KROWS]
        acc_sp = rest[_KROWS]
        idx_sem = rest[_KROWS + 1]
        gsem = rest[_KROWS + 2:2 * _KROWS + 2]
        ssem = rest[2 * _KROWS + 2:3 * _KROWS + 2]
        cid = lax.axis_index("c")
        sid = lax.axis_index("s")
        wid = sid * _NC + cid

        pltpu.sync_copy(zer_hbm.at[pl.ds(sid * rows_per_sub, rows_per_sub)],
                        acc_sp.at[pl.ds(sid * rows_per_sub, rows_per_sub)])
        plsc.subcore_barrier()

        def chunk(g, carry):
            base_row = wid * rows_per_w + g * _KROWS
            c1 = pltpu.async_copy(src_hbm.at[pl.ds(base_row, _KROWS)],
                                  src_v, idx_sem)
            c2 = pltpu.async_copy(dst_hbm.at[pl.ds(base_row, _KROWS)],
                                  dst_v, idx_sem)
            c1.wait()
            c2.wait()
            gs = [pltpu.async_copy(x_hbm.at[src_v.at[j]], bufs[j], gsem[j])
                  for j in range(_KROWS)]
            ss = []
            for j in range(_KROWS):
                gs[j].wait()
                ss.append(pltpu.async_copy(bufs[j], acc_sp.at[dst_v.at[j]],
                                           ssem[j], add=True))
            for cp in ss:
                cp.wait()
            return carry

        lax.fori_loop(0, n_chunks, chunk, 0)
        plsc.subcore_barrier()

        pltpu.sync_copy(
            acc_sp.at[pl.ds(sid * rows_per_sub, rows_per_sub)],
            out_hbm.at[pl.ds(cid * n_pad + sid * rows_per_sub, rows_per_sub)])

    return seg_kernel(x_ext, src2d, dst2d, zeros_init)


def _tc_head(partials, nfp, w1a, w1b, b1r, w2p, b2p, n_pad, blk):
    grid = (n_pad // blk,)

    def head_kernel(p_ref, nf_ref, w1a_ref, w1b_ref, b1_ref, w2_ref, b2_ref,
                    out_ref):
        s = p_ref[0] + p_ref[1]
        deg = jnp.maximum(s[:, 2:3], 1.0)
        agg4 = s / deg
        nf = nf_ref[...]
        hid = jnp.tanh(
            jnp.dot(nf, w1a_ref[...], preferred_element_type=jnp.float32)
            + jnp.dot(agg4, w1b_ref[...], preferred_element_type=jnp.float32)
            + b1_ref[...])
        o = jnp.dot(hid, w2_ref[...], preferred_element_type=jnp.float32) \
            + b2_ref[...]
        mu = jnp.arctan2(o[:, 0:1], o[:, 1:2])
        ang = jnp.mod(mu, 2.0 * np.pi)
        bin_size = 2.0 * np.pi / NUM_CLASSES
        cid = jnp.clip(jnp.floor(ang / bin_size), 0, NUM_CLASSES - 1)
        cid = cid.astype(jnp.int32)
        col = lax.broadcasted_iota(jnp.int32, (blk, NUM_CLASSES), 1)
        out_ref[...] = jnp.where(col == cid, 0.0, -1000.0)

    return pl.pallas_call(
        head_kernel,
        grid=grid,
        in_specs=[
            pl.BlockSpec((2, blk, 8), lambda i: (0, i, 0)),
            pl.BlockSpec((blk, 4), lambda i: (i, 0)),
            pl.BlockSpec((4, 32), lambda i: (0, 0)),
            pl.BlockSpec((8, 32), lambda i: (0, 0)),
            pl.BlockSpec((1, 32), lambda i: (0, 0)),
            pl.BlockSpec((32, NUM_CLASSES), lambda i: (0, 0)),
            pl.BlockSpec((1, NUM_CLASSES), lambda i: (0, 0)),
        ],
        out_specs=pl.BlockSpec((blk, NUM_CLASSES), lambda i: (i, 0)),
        out_shape=jax.ShapeDtypeStruct((n_pad, NUM_CLASSES), jnp.float32),
    )(partials, nfp, w1a, w1b, b1r, w2p, b2p)


def kernel(node_features, edge_index, W1, b1, W2, b2):
    n = node_features.shape[0]
    e = edge_index.shape[1]
    f32 = jnp.float32

    blk = 2048
    n_pad = ((n + blk - 1) // blk) * blk
    chunk_edges = _NW * _KROWS * _ROW
    e_pad = ((e + chunk_edges - 1) // chunk_edges) * chunk_edges
    r = e_pad // _ROW
    rows_per_w = r // _NW

    ones = jnp.ones((n, 1), f32)
    x_ext = jnp.concatenate([node_features[:, 1:3], ones,
                             jnp.zeros((n, 5), f32)], axis=1)
    x_ext = jnp.pad(x_ext, ((0, n_pad - n), (0, 0)))
    ep = jnp.pad(edge_index, ((0, 0), (0, e_pad - e)), constant_values=n)
    src2d = ep[0].reshape(r, _ROW)
    dst2d = ep[1].reshape(r, _ROW)
    zeros_init = jnp.zeros((n_pad, 8), f32)

    partials = _sc_segment_sum(x_ext, src2d, dst2d, zeros_init,
                               n_pad, rows_per_w)
    partials = partials.reshape(2, n_pad, 8)

    nfp = jnp.pad(node_features, ((0, n_pad - n), (0, 1)))
    w1a = jnp.concatenate([W1[0:3], jnp.zeros((1, 32), f32)], axis=0)
    w1b = jnp.concatenate([W1[3:5], jnp.zeros((6, 32), f32)], axis=0)
    b1r = b1.reshape(1, 32)
    w2p = jnp.concatenate([W2[:, 0:2],
                           jnp.zeros((32, NUM_CLASSES - 2), f32)], axis=1)
    b2p = jnp.concatenate([b2[0:2],
                           jnp.zeros((NUM_CLASSES - 2,), f32)]).reshape(1, -1)

    logits = _tc_head(partials, nfp, w1a, w1b, b1r, w2p, b2p, n_pad, blk)
    return logits[:n]


# speedup vs baseline: 53.0350x; 1.0628x over previous
import functools

import jax
import jax.numpy as jnp
import numpy as np
from jax import lax
from jax.experimental import pallas as pl
from jax.experimental.pallas import tpu as pltpu
from jax.experimental.pallas import tpu_sc as plsc

NUM_CLASSES = 8

_NC = 2
_NS = 16
_NW = _NC * _NS
_ROW = 128
_KROWS = 8


def _sc_segment_sum(x_ext, src2d, dst2d, zeros_init, n_pad, rows_per_w):
    rows_per_sub = n_pad // _NS
    n_chunks = rows_per_w // _KROWS
    mesh = plsc.VectorSubcoreMesh(core_axis_name="c", subcore_axis_name="s")

    @functools.partial(
        pl.kernel,
        mesh=mesh,
        out_type=jax.ShapeDtypeStruct((2 * n_pad, 8), jnp.float32),
        scratch_types=(
            [pltpu.VMEM((_KROWS, _ROW), jnp.int32)] * 4
            + [pltpu.VMEM((_ROW, 8), jnp.float32)] * (2 * _KROWS)
            + [pltpu.VMEM_SHARED((n_pad, 8), jnp.float32)]
            + [pltpu.SemaphoreType.DMA] * 6
        ),
        compiler_params=pltpu.CompilerParams(use_tc_tiling_on_sc=False),
    )
    def seg_kernel(x_hbm, src_hbm, dst_hbm, zer_hbm, out_hbm,
                   src_va, dst_va, src_vb, dst_vb, *rest):
        k2 = 2 * _KROWS
        bufs_a = rest[0:_KROWS]
        bufs_b = rest[_KROWS:k2]
        acc_sp = rest[k2]
        idx_sa, idx_sb, gsa, gsb, ssa, ssb = rest[k2 + 1:k2 + 7]
        cid = lax.axis_index("c")
        sid = lax.axis_index("s")
        wid = sid * _NC + cid

        pltpu.sync_copy(zer_hbm.at[pl.ds(sid * rows_per_sub, rows_per_sub)],
                        acc_sp.at[pl.ds(sid * rows_per_sub, rows_per_sub)])
        plsc.subcore_barrier()

        base0 = wid * rows_per_w

        def fire_idx(row0, sv, dv, sem):
            pltpu.async_copy(src_hbm.at[pl.ds(row0, _KROWS)], sv, sem)
            pltpu.async_copy(dst_hbm.at[pl.ds(row0, _KROWS)], dv, sem)

        def drain_idx(sv, dv, sem):
            pltpu.make_async_copy(src_hbm.at[pl.ds(0, _KROWS)], sv, sem).wait()
            pltpu.make_async_copy(dst_hbm.at[pl.ds(0, _KROWS)], dv, sem).wait()

        fire_idx(base0, src_va, dst_va, idx_sa)

        def pair(g, carry):
            row_a = base0 + (2 * g) * _KROWS
            drain_idx(src_va, dst_va, idx_sa)
            cb = []
            pltpu.async_copy(src_hbm.at[pl.ds(row_a + _KROWS, _KROWS)],
                             src_vb, idx_sb)
            pltpu.async_copy(dst_hbm.at[pl.ds(row_a + _KROWS, _KROWS)],
                             dst_vb, idx_sb)
            ga = [pltpu.async_copy(x_hbm.at[src_va.at[j]], bufs_a[j], gsa)
                  for j in range(_KROWS)]
            pltpu.make_async_copy(src_hbm.at[pl.ds(0, _KROWS)], src_vb,
                                  idx_sb).wait()
            pltpu.make_async_copy(src_hbm.at[pl.ds(0, _KROWS)], dst_vb,
                                  idx_sb).wait()
            gb = [pltpu.async_copy(x_hbm.at[src_vb.at[j]], bufs_b[j], gsb)
                  for j in range(_KROWS)]
            for cp in ga:
                cp.wait()
            sa = [pltpu.async_copy(bufs_a[j], acc_sp.at[dst_va.at[j]],
                                   ssa, add=True) for j in range(_KROWS)]
            for cp in gb:
                cp.wait()
            sb = [pltpu.async_copy(bufs_b[j], acc_sp.at[dst_vb.at[j]],
                                   ssb, add=True) for j in range(_KROWS)]
            for cp in sa:
                cp.wait()
            fire_idx(row_a + 2 * _KROWS, src_va, dst_va, idx_sa)
            for cp in sb:
                cp.wait()
            return carry

        lax.fori_loop(0, n_chunks // 2, pair, 0)
        drain_idx(src_va, dst_va, idx_sa)
        plsc.subcore_barrier()

        pltpu.sync_copy(
            acc_sp.at[pl.ds(sid * rows_per_sub, rows_per_sub)],
            out_hbm.at[pl.ds(cid * n_pad + sid * rows_per_sub, rows_per_sub)])

    return seg_kernel(x_ext, src2d, dst2d, zeros_init)


def _tc_head(partials, nfp, w1a, w1b, b1r, w2p, b2p, n_pad, blk):
    grid = (n_pad // blk,)

    def head_kernel(p_ref, nf_ref, w1a_ref, w1b_ref, b1_ref, w2_ref, b2_ref,
                    out_ref):
        s = p_ref[0] + p_ref[1]
        deg = jnp.maximum(s[:, 2:3], 1.0)
        agg4 = s / deg
        nf = nf_ref[...]
        hid = jnp.tanh(
            jnp.dot(nf, w1a_ref[...], preferred_element_type=jnp.float32)
            + jnp.dot(agg4, w1b_ref[...], preferred_element_type=jnp.float32)
            + b1_ref[...])
        o = jnp.dot(hid, w2_ref[...], preferred_element_type=jnp.float32) \
            + b2_ref[...]
        mu = jnp.arctan2(o[:, 0:1], o[:, 1:2])
        ang = jnp.mod(mu, 2.0 * np.pi)
        bin_size = 2.0 * np.pi / NUM_CLASSES
        cid = jnp.clip(jnp.floor(ang / bin_size), 0, NUM_CLASSES - 1)
        cid = cid.astype(jnp.int32)
        col = lax.broadcasted_iota(jnp.int32, (blk, NUM_CLASSES), 1)
        out_ref[...] = jnp.where(col == cid, 0.0, -1000.0)

    return pl.pallas_call(
        head_kernel,
        grid=grid,
        in_specs=[
            pl.BlockSpec((2, blk, 8), lambda i: (0, i, 0)),
            pl.BlockSpec((blk, 4), lambda i: (i, 0)),
            pl.BlockSpec((4, 32), lambda i: (0, 0)),
            pl.BlockSpec((8, 32), lambda i: (0, 0)),
            pl.BlockSpec((1, 32), lambda i: (0, 0)),
            pl.BlockSpec((32, NUM_CLASSES), lambda i: (0, 0)),
            pl.BlockSpec((1, NUM_CLASSES), lambda i: (0, 0)),
        ],
        out_specs=pl.BlockSpec((blk, NUM_CLASSES), lambda i: (i, 0)),
        out_shape=jax.ShapeDtypeStruct((n_pad, NUM_CLASSES), jnp.float32),
    )(partials, nfp, w1a, w1b, b1r, w2p, b2p)


def kernel(node_features, edge_index, W1, b1, W2, b2):
    n = node_features.shape[0]
    e = edge_index.shape[1]
    f32 = jnp.float32

    blk = 2048
    n_pad = ((n + blk - 1) // blk) * blk
    chunk_edges = _NW * _KROWS * _ROW
    e_pad = ((e + chunk_edges - 1) // chunk_edges) * chunk_edges
    r = e_pad // _ROW
    rows_per_w = r // _NW

    ones = jnp.ones((n, 1), f32)
    x_ext = jnp.concatenate([node_features[:, 1:3], ones,
                             jnp.zeros((n, 5), f32)], axis=1)
    x_ext = jnp.pad(x_ext, ((0, n_pad - n), (0, 0)))
    ep = jnp.pad(edge_index, ((0, 0), (0, e_pad - e)), constant_values=n)
    src2d = jnp.pad(ep[0].reshape(r, _ROW), ((0, _KROWS), (0, 0)))
    dst2d = jnp.pad(ep[1].reshape(r, _ROW), ((0, _KROWS), (0, 0)))
    zeros_init = jnp.zeros((n_pad, 8), f32)

    partials = _sc_segment_sum(x_ext, src2d, dst2d, zeros_init,
                               n_pad, rows_per_w)
    partials = partials.reshape(2, n_pad, 8)

    nfp = jnp.pad(node_features, ((0, n_pad - n), (0, 1)))
    w1a = jnp.concatenate([W1[0:3], jnp.zeros((1, 32), f32)], axis=0)
    w1b = jnp.concatenate([W1[3:5], jnp.zeros((6, 32), f32)], axis=0)
    b1r = b1.reshape(1, 32)
    w2p = jnp.concatenate([W2[:, 0:2],
                           jnp.zeros((32, NUM_CLASSES - 2), f32)], axis=1)
    b2p = jnp.concatenate([b2[0:2],
                           jnp.zeros((NUM_CLASSES - 2,), f32)]).reshape(1, -1)

    logits = _tc_head(partials, nfp, w1a, w1b, b1r, w2p, b2p, n_pad, blk)
    return logits[:n]
